# Initial kernel scaffold; baseline (speedup 1.0000x reference)
import math

import jax
import jax.numpy as jnp
from jax import lax
from jax.experimental import pallas as pl
from jax.experimental.pallas import tpu as pltpu
from jax.experimental.pallas import tpu_sc as plsc

D = 32
SCALE = math.sqrt(float(D))
NW = 32
CHUNK = 1024


def _body(x_hbm, table_hbm, out_hbm, idx_v, rows_v, sem):
    nchunk = x_hbm.shape[0] // (NW * CHUNK)
    wid = lax.axis_index("s") * 2 + lax.axis_index("c")
    base = wid * (nchunk * CHUNK)

    @pl.loop(0, nchunk)
    def _chunk(g):
        off = base + g * CHUNK
        pltpu.sync_copy(x_hbm.at[pl.ds(off, CHUNK)], idx_v)
        pltpu.async_copy(table_hbm.at[idx_v], rows_v, sem).wait()

        @pl.loop(0, CHUNK)
        def _row(r):
            rows_v[r, 0:16] = rows_v[r, 0:16] * SCALE
            rows_v[r, 16:32] = rows_v[r, 16:32] * SCALE

        pltpu.sync_copy(rows_v, out_hbm.at[pl.ds(off, CHUNK)])


def kernel(x, table):
    b, t = x.shape
    n = b * t
    xf = x.reshape(n)
    mesh = plsc.VectorSubcoreMesh(core_axis_name="c", subcore_axis_name="s")
    out = pl.kernel(
        _body,
        out_type=jax.ShapeDtypeStruct((n, D), jnp.float32),
        mesh=mesh,
        scratch_types=[
            pltpu.VMEM((CHUNK,), jnp.int32),
            pltpu.VMEM((CHUNK, D), jnp.float32),
            pltpu.SemaphoreType.DMA,
        ],
    )(xf, table)
    return out.reshape(b, t, D)


# speedup vs baseline: 1.2920x; 1.2920x over previous
import math

import jax
import jax.numpy as jnp
from jax import lax
from jax.experimental import pallas as pl
from jax.experimental.pallas import tpu as pltpu
from jax.experimental.pallas import tpu_sc as plsc

D = 32
SCALE = math.sqrt(float(D))
NW = 32
CHUNK = 1024


def _body(x_hbm, table_hbm, out_hbm, idx_v, rows_v, sem):
    nchunk = x_hbm.shape[0] // (NW * CHUNK)
    wid = lax.axis_index("s") * 2 + lax.axis_index("c")
    base = wid * (nchunk * CHUNK)

    @pl.loop(0, nchunk)
    def _chunk(g):
        off = base + g * CHUNK
        pltpu.sync_copy(x_hbm.at[pl.ds(off, CHUNK)], idx_v)
        pltpu.async_copy(table_hbm.at[idx_v], rows_v, sem).wait()

        @pl.loop(0, CHUNK)
        def _row(r):
            rows_v[r, 0:16] = rows_v[r, 0:16] * SCALE
            rows_v[r, 16:32] = rows_v[r, 16:32] * SCALE

        pltpu.sync_copy(rows_v, out_hbm.at[pl.ds(off, CHUNK)])


def kernel(x, table):
    b, t = x.shape
    n = b * t
    xf = x.reshape(n)
    mesh = plsc.VectorSubcoreMesh(core_axis_name="c", subcore_axis_name="s")
    out = pl.kernel(
        _body,
        out_type=jax.ShapeDtypeStruct((n, D), jnp.float32),
        mesh=mesh,
        scratch_types=[
            pltpu.VMEM((CHUNK,), jnp.int32),
            pltpu.VMEM((CHUNK, D), jnp.float32),
            pltpu.SemaphoreType.DMA,
        ],
        compiler_params=pltpu.CompilerParams(use_tc_tiling_on_sc=False),
    )(xf, table)
    return out.reshape(b, t, D)


# speedup vs baseline: 1.4683x; 1.1365x over previous
import math

import jax
import jax.numpy as jnp
from jax import lax
from jax.experimental import pallas as pl
from jax.experimental.pallas import tpu as pltpu
from jax.experimental.pallas import tpu_sc as plsc

D = 32
SCALE = math.sqrt(float(D))
NW = 32
CHUNK = 1280


def _body(x_hbm, table_hbm, out_hbm, idx0, idx1, rows0, rows1, sem0, sem1):
    nchunk = x_hbm.shape[0] // (NW * CHUNK)
    wid = lax.axis_index("s") * 2 + lax.axis_index("c")
    base = wid * (nchunk * CHUNK)
    bufs = ((idx0, rows0, sem0), (idx1, rows1, sem1))

    pltpu.sync_copy(x_hbm.at[pl.ds(base, CHUNK)], idx0)
    pltpu.async_copy(table_hbm.at[idx0], rows0, sem0)

    @pl.loop(0, nchunk // 2)
    def _pair(gg):
        for b in range(2):
            g = gg * 2 + b
            idx_c, rows_c, sem_c = bufs[b]
            idx_n, rows_n, sem_n = bufs[1 - b]

            @pl.when(g + 1 < nchunk)
            def _():
                off_n = base + (g + 1) * CHUNK
                pltpu.sync_copy(x_hbm.at[pl.ds(off_n, CHUNK)], idx_n)
                pltpu.async_copy(table_hbm.at[idx_n], rows_n, sem_n)

            pltpu.make_async_copy(table_hbm.at[idx_c], rows_c, sem_c).wait()

            @plsc.parallel_loop(0, CHUNK, unroll=8)
            def _row(r):
                rows_c[r, 0:16] = rows_c[r, 0:16] * SCALE
                rows_c[r, 16:32] = rows_c[r, 16:32] * SCALE

            off = base + g * CHUNK
            pltpu.sync_copy(rows_c, out_hbm.at[pl.ds(off, CHUNK)])


def kernel(x, table):
    b, t = x.shape
    n = b * t
    xf = x.reshape(n)
    mesh = plsc.VectorSubcoreMesh(core_axis_name="c", subcore_axis_name="s")
    out = pl.kernel(
        _body,
        out_type=jax.ShapeDtypeStruct((n, D), jnp.float32),
        mesh=mesh,
        scratch_types=[
            pltpu.VMEM((CHUNK,), jnp.int32),
            pltpu.VMEM((CHUNK,), jnp.int32),
            pltpu.VMEM((CHUNK, D), jnp.float32),
            pltpu.VMEM((CHUNK, D), jnp.float32),
            pltpu.SemaphoreType.DMA,
            pltpu.SemaphoreType.DMA,
        ],
        compiler_params=pltpu.CompilerParams(use_tc_tiling_on_sc=False),
    )(xf, table)
    return out.reshape(b, t, D)
